# baseline (speedup 1.0000x reference)
import jax
import jax.numpy as jnp
from jax.experimental import pallas as pl
from jax.experimental.pallas import tpu as pltpu

D = 2048
NE = 8
ED = 8192
KE = 512
NS1 = ED // KE
DB = 256

_NT = (((1,), (1,)), ((), ()))


def _up_kernel(x_ref, wg_ref, wup_ref, h_ref, scale_ref, xb_ref):
    @pl.when(pl.program_id(0) == 0)
    def _gate():
        xb = x_ref[...].astype(jnp.bfloat16)
        xb_ref[...] = xb
        lt = jax.lax.dot_general(
            wg_ref[...], xb, _NT, preferred_element_type=jnp.float32
        )
        m1 = lt[0:1, :]
        m2 = jnp.full_like(m1, -jnp.inf)
        for e in range(1, NE):
            le = lt[e : e + 1, :]
            gt = le > m1
            m2 = jnp.where(gt, m1, jnp.maximum(m2, le))
            m1 = jnp.maximum(m1, le)
        den = jnp.zeros_like(m1)
        for e in range(NE):
            den = den + jnp.exp(lt[e : e + 1, :] - m1)
        s = (1.0 + jnp.exp(m2 - m1)) / den
        scale_ref[...] = s / (s + 1e-9)

    h = jax.lax.dot_general(
        xb_ref[...],
        wup_ref[...].astype(jnp.bfloat16),
        _NT,
        preferred_element_type=jnp.float32,
    )
    h_ref[...] = (h * jax.lax.logistic(h)).astype(jnp.bfloat16)


def _down_kernel(h_ref, wdn_ref, scale_ref, outt_ref):
    y = jax.lax.dot_general(
        wdn_ref[...].astype(jnp.bfloat16),
        h_ref[...],
        _NT,
        preferred_element_type=jnp.float32,
    )
    outt_ref[...] = y * scale_ref[...]


@jax.jit
def kernel(x, W_gate, W_up, W_down):
    B, S, Dm = x.shape
    T = B * S
    xf = x.reshape(T, Dm)
    wg = W_gate.astype(jnp.bfloat16)

    h, scale = pl.pallas_call(
        _up_kernel,
        grid=(NS1,),
        in_specs=[
            pl.BlockSpec((T, Dm), lambda i: (0, 0)),
            pl.BlockSpec((NE, Dm), lambda i: (0, 0)),
            pl.BlockSpec((KE, Dm), lambda i: (i, 0)),
        ],
        out_specs=[
            pl.BlockSpec((T, KE), lambda i: (0, i)),
            pl.BlockSpec((1, T), lambda i: (0, 0)),
        ],
        out_shape=[
            jax.ShapeDtypeStruct((T, ED), jnp.bfloat16),
            jax.ShapeDtypeStruct((1, T), jnp.float32),
        ],
        scratch_shapes=[pltpu.VMEM((T, Dm), jnp.bfloat16)],
        compiler_params=pltpu.CompilerParams(
            dimension_semantics=("arbitrary",),
        ),
    )(xf, wg, W_up)

    outt = pl.pallas_call(
        _down_kernel,
        grid=(Dm // DB,),
        in_specs=[
            pl.BlockSpec((T, ED), lambda i: (0, 0)),
            pl.BlockSpec((DB, ED), lambda i: (i, 0)),
            pl.BlockSpec((1, T), lambda i: (0, 0)),
        ],
        out_specs=pl.BlockSpec((DB, T), lambda i: (i, 0)),
        out_shape=jax.ShapeDtypeStruct((Dm, T), jnp.float32),
        compiler_params=pltpu.CompilerParams(
            dimension_semantics=("arbitrary",),
        ),
    )(h, W_down, scale)
    return outt.T.reshape(B, S, Dm)


# speedup vs baseline: 1.1538x; 1.1538x over previous
import jax
import jax.numpy as jnp
from jax.experimental import pallas as pl
from jax.experimental.pallas import tpu as pltpu

D = 2048
NE = 8
ED = 8192
KE = 512
NS1 = ED // KE
DB = 256

_NT = (((1,), (1,)), ((), ()))


def _up_kernel(x_ref, wg_ref, wup_ref, h_ref, scale_ref, xb_ref):
    @pl.when(pl.program_id(0) == 0)
    def _gate():
        xb = x_ref[...].astype(jnp.bfloat16)
        xb_ref[...] = xb
        lt = jax.lax.dot_general(
            wg_ref[...], xb, _NT, preferred_element_type=jnp.float32
        )
        m1 = lt[0:1, :]
        m2 = jnp.full_like(m1, -jnp.inf)
        for e in range(1, NE):
            le = lt[e : e + 1, :]
            gt = le > m1
            m2 = jnp.where(gt, m1, jnp.maximum(m2, le))
            m1 = jnp.maximum(m1, le)
        den = jnp.zeros_like(m1)
        for e in range(NE):
            den = den + jnp.exp(lt[e : e + 1, :] - m1)
        s = (1.0 + jnp.exp(m2 - m1)) / den
        scale_ref[...] = s / (s + 1e-9)

    h = jax.lax.dot_general(
        xb_ref[...],
        wup_ref[...].astype(jnp.bfloat16),
        _NT,
        preferred_element_type=jnp.float32,
    )
    h_ref[...] = (h * jax.lax.logistic(h)).astype(jnp.bfloat16)


def _down_kernel(h_ref, wdn_ref, scale_ref, out_ref):
    y = jax.lax.dot_general(
        wdn_ref[...].astype(jnp.bfloat16),
        h_ref[...],
        _NT,
        preferred_element_type=jnp.float32,
    )
    out_ref[...] = jax.lax.transpose(y * scale_ref[...], (1, 0))


@jax.jit
def kernel(x, W_gate, W_up, W_down):
    B, S, Dm = x.shape
    T = B * S
    xf = x.reshape(T, Dm)
    wg = W_gate.astype(jnp.bfloat16)

    h, scale = pl.pallas_call(
        _up_kernel,
        grid=(NS1,),
        in_specs=[
            pl.BlockSpec((T, Dm), lambda i: (0, 0)),
            pl.BlockSpec((NE, Dm), lambda i: (0, 0)),
            pl.BlockSpec((KE, Dm), lambda i: (i, 0)),
        ],
        out_specs=[
            pl.BlockSpec((T, KE), lambda i: (0, i)),
            pl.BlockSpec((1, T), lambda i: (0, 0)),
        ],
        out_shape=[
            jax.ShapeDtypeStruct((T, ED), jnp.bfloat16),
            jax.ShapeDtypeStruct((1, T), jnp.float32),
        ],
        scratch_shapes=[pltpu.VMEM((T, Dm), jnp.bfloat16)],
        compiler_params=pltpu.CompilerParams(
            dimension_semantics=("arbitrary",),
        ),
    )(xf, wg, W_up)

    outt = pl.pallas_call(
        _down_kernel,
        grid=(Dm // DB,),
        in_specs=[
            pl.BlockSpec((T, ED), lambda i: (0, 0)),
            pl.BlockSpec((DB, ED), lambda i: (i, 0)),
            pl.BlockSpec((1, T), lambda i: (0, 0)),
        ],
        out_specs=pl.BlockSpec((T, DB), lambda i: (0, i)),
        out_shape=jax.ShapeDtypeStruct((T, Dm), jnp.float32),
        compiler_params=pltpu.CompilerParams(
            dimension_semantics=("arbitrary",),
        ),
    )(h, W_down, scale)
    return outt.reshape(B, S, Dm)
